# baseline (speedup 1.0000x reference)
import functools

import jax
import jax.numpy as jnp
from jax import lax
from jax.experimental import pallas as pl
from jax.experimental.pallas import tpu as pltpu
from jax.experimental.pallas import tpu_sc as plsc

N_ENT = 100000
N_REL = 64
D = 128
K = 16
B = 1024
N_HEADS = 2

NC, NS, L = 2, 16, 16
NW = NC * NS
EPW = B // NW
KK = K * K


def _norm_body(x_ref, o_ref):
    x = x_ref[...]
    xx = x * x
    n2 = jnp.dot(xx, jnp.ones((D, 1), jnp.float32),
                 preferred_element_type=jnp.float32)
    o_ref[...] = x * jnp.where(n2 > 1.0, lax.rsqrt(n2), 1.0)


def _rownorm(x):
    n2 = jnp.sum(x * x, axis=1, keepdims=True)
    return x * jnp.where(n2 > 1.0, lax.rsqrt(n2), 1.0)


def _normalize_table(t, blk):
    rows = t.shape[0]
    return pl.pallas_call(
        _norm_body,
        grid=(rows // blk,),
        in_specs=[pl.BlockSpec((blk, D), lambda i: (i, 0))],
        out_specs=pl.BlockSpec((blk, D), lambda i: (i, 0)),
        out_shape=jax.ShapeDtypeStruct((rows, D), jnp.float32),
    )(t)


_IOTA = None


def _extract16(rows_v, parents_v, p, iota16):
    ev = plsc.load_gather(parents_v, [jnp.full((L,), p, jnp.int32)])
    col = ((ev & 7) << 4) + iota16
    return plsc.load_gather(rows_v, [jnp.full((L,), p, jnp.int32), col])


def _phase_a_body(eidx_hbm, adje_hbm, adjr_hbm, traw_hbm,
                  ne1_out, nr1_out, ne2_out, nr2_out, h_out, t1_out,
                  eidx_v, ae0_v, ar0_v, ae1_v, ar1_v,
                  ne1_v, nr1_v, ne2_v, nr2_v, h_v, t10_v, t11_v,
                  sem0, sem1):
    wid = lax.axis_index("s") * NC + lax.axis_index("c")
    base = wid * EPW
    pltpu.sync_copy(eidx_hbm.at[pl.ds(base, EPW)], eidx_v)
    pltpu.async_copy(adje_hbm.at[eidx_v], ae0_v.at[pl.ds(0, EPW)], sem0)
    pltpu.async_copy(adjr_hbm.at[eidx_v], ar0_v.at[pl.ds(0, EPW)], sem0)
    pltpu.async_copy(traw_hbm.at[eidx_v], h_v, sem0)
    pltpu.make_async_copy(adje_hbm.at[eidx_v], ae0_v.at[pl.ds(0, EPW)],
                          sem0).wait()
    pltpu.make_async_copy(adjr_hbm.at[eidx_v], ar0_v.at[pl.ds(0, EPW)],
                          sem0).wait()

    @pl.loop(0, EPW)
    def flatten1(p):
        ne1_v[pl.ds(K * p, K)] = ae0_v[p, :]
        nr1_v[pl.ds(K * p, K)] = ar0_v[p, :]
    pltpu.sync_copy(ne1_v, ne1_out.at[pl.ds(base * K, EPW * K)])
    pltpu.sync_copy(nr1_v, nr1_out.at[pl.ds(base * K, EPW * K)])
    pltpu.make_async_copy(traw_hbm.at[eidx_v], h_v, sem0).wait()
    pltpu.sync_copy(h_v, h_out.at[pl.ds(base, EPW)])

    bufs = ((ae0_v, ar0_v, t10_v, sem0), (ae1_v, ar1_v, t11_v, sem1))

    def fire(c, bf):
        ae, ar, t1, sem = bf
        idx = ne1_v.at[pl.ds(128 * c, 128)]
        pltpu.async_copy(adje_hbm.at[idx], ae, sem)
        pltpu.async_copy(adjr_hbm.at[idx], ar, sem)
        pltpu.async_copy(traw_hbm.at[idx], t1, sem)

    def wait(c, bf):
        ae, ar, t1, sem = bf
        idx = ne1_v.at[pl.ds(128 * c, 128)]
        pltpu.make_async_copy(adje_hbm.at[idx], ae, sem).wait()
        pltpu.make_async_copy(adjr_hbm.at[idx], ar, sem).wait()
        pltpu.make_async_copy(traw_hbm.at[idx], t1, sem).wait()

    fire(0, bufs[0])
    for c in range(4):
        bf = bufs[c % 2]
        wait(c, bf)
        if c + 1 < 4:
            fire(c + 1, bufs[(c + 1) % 2])
        ae, ar, t1, _ = bf

        @pl.loop(0, 128)
        def flatten2(p, _ae=ae, _ar=ar):
            ne2_v[pl.ds(K * p, K)] = _ae[p, :]
            nr2_v[pl.ds(K * p, K)] = _ar[p, :]
        row0 = (base * K + 128 * c)
        pltpu.sync_copy(ne2_v, ne2_out.at[pl.ds(row0 * K, 128 * K)])
        pltpu.sync_copy(nr2_v, nr2_out.at[pl.ds(row0 * K, 128 * K)])
        pltpu.sync_copy(t1, t1_out.at[pl.ds(row0, 128)])


def _phase_a(entity_idx, adje, adjr, traw):
    mesh = plsc.VectorSubcoreMesh(core_axis_name="c", subcore_axis_name="s",
                                  num_cores=NC, num_subcores=NS)
    fn = pl.kernel(
        _phase_a_body,
        out_type=[
            jax.ShapeDtypeStruct((B * K,), jnp.int32),
            jax.ShapeDtypeStruct((B * K,), jnp.int32),
            jax.ShapeDtypeStruct((B * KK,), jnp.int32),
            jax.ShapeDtypeStruct((B * KK,), jnp.int32),
            jax.ShapeDtypeStruct((B, D), jnp.float32),
            jax.ShapeDtypeStruct((B * K, D), jnp.float32),
        ],
        mesh=mesh,
        scratch_types=[
            pltpu.VMEM((EPW,), jnp.int32),
            pltpu.VMEM((128, K), jnp.int32),
            pltpu.VMEM((128, K), jnp.int32),
            pltpu.VMEM((128, K), jnp.int32),
            pltpu.VMEM((128, K), jnp.int32),
            pltpu.VMEM((EPW * K,), jnp.int32),
            pltpu.VMEM((EPW * K,), jnp.int32),
            pltpu.VMEM((128 * K,), jnp.int32),
            pltpu.VMEM((128 * K,), jnp.int32),
            pltpu.VMEM((EPW, D), jnp.float32),
            pltpu.VMEM((128, D), jnp.float32),
            pltpu.VMEM((128, D), jnp.float32),
            pltpu.SemaphoreType.DMA,
            pltpu.SemaphoreType.DMA,
        ],
        compiler_params=pltpu.CompilerParams(
            needs_layout_passes=False, use_tc_tiling_on_sc=False),
    )
    return fn(entity_idx, adje, adjr, traw)


def _softmax_weights(rel_ref, at_ref, w_ref):
    m = jnp.full((L,), -3.0e38, jnp.float32)
    for g in range(KK // L):
        rv = rel_ref[pl.ds(L * g, L)]
        av = plsc.load_gather(at_ref, [rv])
        w_ref[pl.ds(L * g, L)] = av
        m = jnp.maximum(m, av)
    mm = jnp.full((L,), lax.reduce_max(m, axes=(0,)), jnp.float32)
    s = jnp.zeros((L,), jnp.float32)
    for g in range(KK // L):
        ev = jnp.exp(w_ref[pl.ds(L * g, L)] - mm)
        w_ref[pl.ds(L * g, L)] = ev
        s = s + ev
    sb = jnp.full((L,), lax.reduce_sum(s, axes=(0,)), jnp.float32)
    for g in range(KK // L):
        w_ref[pl.ds(L * g, L)] = w_ref[pl.ds(L * g, L)] / sb


def _accumulate(rows_ref, w_ref, half, accs):
    @pl.loop(0, 128, init_carry=accs, unroll=4)
    def row_body(r, accs):
        wbr = plsc.load_gather(
            w_ref, [jnp.full((L,), 128 * half + r, jnp.int32)])
        return tuple(a + wbr * rows_ref[r, pl.ds(L * j, L)]
                     for j, a in enumerate(accs))
    return row_body


def _phase_c_body(boff, nb, ne2_hbm, nr2_hbm, at1f_hbm, tn_hbm, s2_hbm,
                  idx0_v, idx1_v, rel0_v, rel1_v, at0_v, at1_v,
                  w0_v, w1_v, r00_v, r01_v, r10_v, r11_v, acc_v,
                  semi0, semi1, semr00, semr01, semr10, semr11):
    wid = lax.axis_index("s") * NC + lax.axis_index("c")
    eph = nb // NW
    base = boff + wid * eph
    NP = eph // 2

    def stage(b, idx_v, rel_v, at_v, sem):
        c0 = pltpu.async_copy(ne2_hbm.at[pl.ds(b * KK, KK)], idx_v, sem)
        c1 = pltpu.async_copy(nr2_hbm.at[pl.ds(b * KK, KK)], rel_v, sem)
        c2 = pltpu.async_copy(
            at1f_hbm.at[pl.ds((b - boff) * N_REL, N_REL)], at_v, sem)
        return c0, c1, c2

    def wait_stage(b, idx_v, rel_v, at_v, sem):
        pltpu.make_async_copy(ne2_hbm.at[pl.ds(b * KK, KK)], idx_v, sem).wait()
        pltpu.make_async_copy(nr2_hbm.at[pl.ds(b * KK, KK)], rel_v, sem).wait()
        pltpu.make_async_copy(
            at1f_hbm.at[pl.ds((b - boff) * N_REL, N_REL)], at_v, sem).wait()

    def fire_rows(idx_v, ra_v, rb_v, sa, sb):
        pltpu.async_copy(tn_hbm.at[idx_v.at[pl.ds(0, 128)]], ra_v, sa)
        pltpu.async_copy(tn_hbm.at[idx_v.at[pl.ds(128, 128)]], rb_v, sb)

    def wait_rows(idx_v, ra_v, rb_v, sa, sb):
        pltpu.make_async_copy(tn_hbm.at[idx_v.at[pl.ds(0, 128)]], ra_v,
                              sa).wait()
        pltpu.make_async_copy(tn_hbm.at[idx_v.at[pl.ds(128, 128)]], rb_v,
                              sb).wait()

    def reduce_store(b, ra_v, rb_v, w_v):
        accs = tuple(jnp.zeros((L,), jnp.float32) for _ in range(D // L))
        accs = _accumulate(ra_v, w_v, 0, accs)
        accs = _accumulate(rb_v, w_v, 1, accs)
        for j in range(D // L):
            acc_v[pl.ds(L * j, L)] = accs[j]
        pltpu.sync_copy(acc_v, s2_hbm.at[pl.ds((b - boff) * D, D)])

    stage(base, idx0_v, rel0_v, at0_v, semi0)
    wait_stage(base, idx0_v, rel0_v, at0_v, semi0)
    fire_rows(idx0_v, r00_v, r01_v, semr00, semr01)
    stage(base + 1, idx1_v, rel1_v, at1_v, semi1)

    @pl.loop(0, NP)
    def pair(t):
        b0 = base + 2 * t
        wait_stage(b0 + 1, idx1_v, rel1_v, at1_v, semi1)
        fire_rows(idx1_v, r10_v, r11_v, semr10, semr11)
        _softmax_weights(rel0_v, at0_v, w0_v)
        wait_rows(idx0_v, r00_v, r01_v, semr00, semr01)

        @pl.when(t + 1 < NP)
        def _():
            stage(b0 + 2, idx0_v, rel0_v, at0_v, semi0)
        reduce_store(b0, r00_v, r01_v, w0_v)

        _softmax_weights(rel1_v, at1_v, w1_v)

        @pl.when(t + 1 < NP)
        def _():
            wait_stage(b0 + 2, idx0_v, rel0_v, at0_v, semi0)
            fire_rows(idx0_v, r00_v, r01_v, semr00, semr01)
        wait_rows(idx1_v, r10_v, r11_v, semr10, semr11)

        @pl.when(t + 1 < NP)
        def _():
            stage(b0 + 3, idx1_v, rel1_v, at1_v, semi1)
        reduce_store(b0 + 1, r10_v, r11_v, w1_v)


def _phase_c(ne2f, nr2f, at1f, tn, boff=0, nb=B):
    mesh = plsc.VectorSubcoreMesh(core_axis_name="c", subcore_axis_name="s",
                                  num_cores=NC, num_subcores=NS)
    fn = pl.kernel(
        functools.partial(_phase_c_body, boff, nb),
        out_type=jax.ShapeDtypeStruct((nb * D,), jnp.float32),
        mesh=mesh,
        scratch_types=[
            pltpu.VMEM((KK,), jnp.int32),
            pltpu.VMEM((KK,), jnp.int32),
            pltpu.VMEM((KK,), jnp.int32),
            pltpu.VMEM((KK,), jnp.int32),
            pltpu.VMEM((N_REL,), jnp.float32),
            pltpu.VMEM((N_REL,), jnp.float32),
            pltpu.VMEM((KK,), jnp.float32),
            pltpu.VMEM((KK,), jnp.float32),
            pltpu.VMEM((128, D), jnp.float32),
            pltpu.VMEM((128, D), jnp.float32),
            pltpu.VMEM((128, D), jnp.float32),
            pltpu.VMEM((128, D), jnp.float32),
            pltpu.VMEM((D,), jnp.float32),
            pltpu.SemaphoreType.DMA,
            pltpu.SemaphoreType.DMA,
            pltpu.SemaphoreType.DMA,
            pltpu.SemaphoreType.DMA,
            pltpu.SemaphoreType.DMA,
            pltpu.SemaphoreType.DMA,
        ],
        compiler_params=pltpu.CompilerParams(needs_layout_passes=False),
    )
    return fn(ne2f, nr2f, at1f, tn)


_BS_B = 128


def _leaky(x):
    return jnp.where(x >= 0.0, x, 0.2 * x)


def _sigmoid(x):
    return 1.0 / (1.0 + jnp.exp(-x))


def _att_table(base, a1t, ra1r, a2, a3t, bs):
    pre = jnp.dot(base, a1t, preferred_element_type=jnp.float32)
    x = jnp.maximum(pre[:, None, :] + ra1r[None, :, :], 0.0)
    x = x.reshape(bs * N_REL, D)
    x = jnp.maximum(jnp.dot(x, a2, preferred_element_type=jnp.float32), 0.0)
    att = jnp.sum(x * a3t, axis=1)
    return _sigmoid(att).reshape(bs, N_REL)


def _onehot_pick(tbl, idx):
    out = jnp.zeros(idx.shape, jnp.float32)
    for r in range(N_REL):
        out = out + jnp.where(idx == r, tbl[:, r:r + 1], 0.0)
    return out


def _softmax_last(x):
    m = jnp.max(x, axis=-1, keepdims=True)
    e = jnp.exp(x - m)
    return e / jnp.sum(e, axis=-1, keepdims=True)


def _phase_b_body(h_ref, t1_ref, nr1_ref, rel_ref,
                  a1_ref, a2_ref, a3t_ref, wx00_ref, wx01_ref,
                  wxb00_ref, wxb01_ref, w1w_ref, w1b_ref, w2w_ref, w2b_ref,
                  at1_ref, base1_ref, ee0_ref):
    bs = _BS_B
    h = _rownorm(h_ref[...])
    t1 = _rownorm(t1_ref[...]).reshape(bs, K, D)
    reln = _rownorm(rel_ref[...])
    a1t = a1_ref[:D, :]
    ra1r = jnp.dot(reln, a1_ref[D:, :], preferred_element_type=jnp.float32)
    a2 = a2_ref[...]
    a3t = a3t_ref[...]

    base1 = jnp.sum(t1, axis=1)
    base1_ref[...] = base1

    at0 = _att_table(h, a1t, ra1r, a2, a3t, bs)
    at1 = _att_table(base1, a1t, ra1r, a2, a3t, bs)

    att1 = _onehot_pick(at0, nr1_ref[...])
    attn1 = _softmax_last(att1)
    vec1 = jnp.zeros((bs, D), jnp.float32)
    for k in range(K):
        vec1 = vec1 + attn1[:, k:k + 1] * t1[:, k, :]
    e0 = jnp.dot(vec1, wx00_ref[...], preferred_element_type=jnp.float32) + wxb00_ref[...]
    e1 = jnp.dot(vec1, wx01_ref[...], preferred_element_type=jnp.float32) + wxb01_ref[...]
    vec = _leaky(jnp.concatenate([e0, e1], axis=-1))
    hexp = jnp.concatenate([h, h], axis=-1)
    ee0 = (_leaky(jnp.dot(hexp + vec, w1w_ref[...], preferred_element_type=jnp.float32) + w1b_ref[...])
           + _leaky(jnp.dot(hexp * vec, w2w_ref[...], preferred_element_type=jnp.float32) + w2b_ref[...]))
    ee0_ref[...] = ee0
    at1_ref[...] = at1


def _phase_b(h_norm, t1n, nr1, relation_table,
             A1, A2, A3, Wx, Wx_b, W1_w, W1_b, W2_w, W2_b):
    bs = _BS_B
    nb = h_norm.shape[0]
    grid = nb // bs
    full = lambda shape: pl.BlockSpec(shape, lambda i: tuple(0 for _ in shape))
    out = pl.pallas_call(
        _phase_b_body,
        grid=(grid,),
        in_specs=[
            pl.BlockSpec((bs, D), lambda i: (i, 0)),
            pl.BlockSpec((bs * K, D), lambda i: (i, 0)),
            pl.BlockSpec((bs, K), lambda i: (i, 0)),
            full((N_REL, D)),
            full((2 * D, D)),
            full((D, D)),
            full((1, D)),
            full((D, D)),
            full((D, D)),
            full((1, D)),
            full((1, D)),
            full((2 * D, 2 * D)),
            full((1, 2 * D)),
            full((2 * D, 2 * D)),
            full((1, 2 * D)),
        ],
        out_specs=[
            pl.BlockSpec((bs, N_REL), lambda i: (i, 0)),
            pl.BlockSpec((bs, D), lambda i: (i, 0)),
            pl.BlockSpec((bs, 2 * D), lambda i: (i, 0)),
        ],
        out_shape=[
            jax.ShapeDtypeStruct((nb, N_REL), jnp.float32),
            jax.ShapeDtypeStruct((nb, D), jnp.float32),
            jax.ShapeDtypeStruct((nb, 2 * D), jnp.float32),
        ],
    )(h_norm, t1n, nr1, relation_table,
      A1, A2, A3.reshape(1, D), Wx[0, 0], Wx[0, 1],
      Wx_b[0, 0].reshape(1, D), Wx_b[0, 1].reshape(1, D),
      W1_w, W1_b.reshape(1, 2 * D), W2_w, W2_b.reshape(1, 2 * D))
    return out


_BS_D = 128


def _phase_d_body(s2_ref, base1_ref, ee0_ref, h_ref,
                  wx10_ref, wx11_ref, wxb10_ref, wxb11_ref,
                  w1w_ref, w1b_ref, w2w_ref, w2b_ref, out_ref):
    s2 = s2_ref[...]
    base1 = base1_ref[...]
    e0 = jnp.dot(s2, wx10_ref[...], preferred_element_type=jnp.float32) + wxb10_ref[...]
    e1 = jnp.dot(s2, wx11_ref[...], preferred_element_type=jnp.float32) + wxb11_ref[...]
    vec = _leaky(jnp.concatenate([e0, e1], axis=-1))
    hexp = jnp.concatenate([base1, base1], axis=-1)
    ee1 = (_leaky(jnp.dot(hexp + vec, w1w_ref[...], preferred_element_type=jnp.float32) + w1b_ref[...])
           + _leaky(jnp.dot(hexp * vec, w2w_ref[...], preferred_element_type=jnp.float32) + w2b_ref[...]))
    out_ref[...] = jnp.concatenate([ee1, ee0_ref[...], _rownorm(h_ref[...])],
                                   axis=-1)


def _phase_d(s2, base1, ee0, h_norm, Wx, Wx_b, W1_w, W1_b, W2_w, W2_b):
    bs = _BS_D
    nb = s2.shape[0]
    grid = nb // bs
    full = lambda shape: pl.BlockSpec(shape, lambda i: tuple(0 for _ in shape))
    return pl.pallas_call(
        _phase_d_body,
        grid=(grid,),
        in_specs=[
            pl.BlockSpec((bs, D), lambda i: (i, 0)),
            pl.BlockSpec((bs, D), lambda i: (i, 0)),
            pl.BlockSpec((bs, 2 * D), lambda i: (i, 0)),
            pl.BlockSpec((bs, D), lambda i: (i, 0)),
            full((D, D)),
            full((D, D)),
            full((1, D)),
            full((1, D)),
            full((2 * D, 2 * D)),
            full((1, 2 * D)),
            full((2 * D, 2 * D)),
            full((1, 2 * D)),
        ],
        out_specs=pl.BlockSpec((bs, 5 * D), lambda i: (i, 0)),
        out_shape=jax.ShapeDtypeStruct((nb, 5 * D), jnp.float32),
    )(s2, base1, ee0, h_norm, Wx[1, 0], Wx[1, 1],
      Wx_b[1, 0].reshape(1, D), Wx_b[1, 1].reshape(1, D),
      W1_w, W1_b.reshape(1, 2 * D), W2_w, W2_b.reshape(1, 2 * D))


def kernel(entity_idx, adj_entity, adj_relation, entity_table, relation_table,
           A1, A2, A3, Wx, Wx_b, W1_w, W1_b, W2_w, W2_b):
    tn = _normalize_table(entity_table, blk=4000)
    ne1f, nr1f, ne2f, nr2f, h_raw, t1_raw = _phase_a(
        entity_idx, adj_entity, adj_relation, entity_table)
    H = B // 4
    nr1r = nr1f.reshape(B, K)
    outs = []
    parts = []
    for half in range(4):
        sl = slice(half * H, (half + 1) * H)
        at1, base1, ee0 = _phase_b(
            h_raw[sl], t1_raw[half * H * K:(half + 1) * H * K], nr1r[sl],
            relation_table, A1, A2, A3, Wx, Wx_b, W1_w, W1_b, W2_w, W2_b)
        parts.append((sl, at1, base1, ee0))
    for half, (sl, at1, base1, ee0) in enumerate(parts):
        s2 = _phase_c(ne2f, nr2f, at1.reshape(-1), tn,
                      boff=half * H, nb=H).reshape(H, D)
        outs.append(_phase_d(s2, base1, ee0, h_raw[sl],
                             Wx, Wx_b, W1_w, W1_b, W2_w, W2_b))
    return jnp.concatenate(outs, axis=0)


# speedup vs baseline: 1.0098x; 1.0098x over previous
import functools

import jax
import jax.numpy as jnp
from jax import lax
from jax.experimental import pallas as pl
from jax.experimental.pallas import tpu as pltpu
from jax.experimental.pallas import tpu_sc as plsc

N_ENT = 100000
N_REL = 64
D = 128
K = 16
B = 1024
N_HEADS = 2

NC, NS, L = 2, 16, 16
NW = NC * NS
EPW = B // NW
KK = K * K


def _norm_body(x_ref, o_ref):
    x = x_ref[...]
    xx = x * x
    n2 = jnp.dot(xx, jnp.ones((D, 1), jnp.float32),
                 preferred_element_type=jnp.float32)
    o_ref[...] = x * jnp.where(n2 > 1.0, lax.rsqrt(n2), 1.0)


def _rownorm(x):
    n2 = jnp.sum(x * x, axis=1, keepdims=True)
    return x * jnp.where(n2 > 1.0, lax.rsqrt(n2), 1.0)


def _normalize_table(t, blk):
    rows = t.shape[0]
    return pl.pallas_call(
        _norm_body,
        grid=(rows // blk,),
        in_specs=[pl.BlockSpec((blk, D), lambda i: (i, 0))],
        out_specs=pl.BlockSpec((blk, D), lambda i: (i, 0)),
        out_shape=jax.ShapeDtypeStruct((rows, D), jnp.float32),
    )(t)


_IOTA = None


def _extract16(rows_v, parents_v, p, iota16):
    ev = plsc.load_gather(parents_v, [jnp.full((L,), p, jnp.int32)])
    col = ((ev & 7) << 4) + iota16
    return plsc.load_gather(rows_v, [jnp.full((L,), p, jnp.int32), col])


def _phase_a_body(eidx_hbm, adje_hbm, adjr_hbm, traw_hbm,
                  ne1_out, nr1_out, ne2_out, nr2_out, h_out, t1_out,
                  eidx_v, ae0_v, ar0_v, ae1_v, ar1_v,
                  ne1_v, nr1_v, ne2_v, nr2_v, h_v, t10_v, t11_v,
                  sem0, sem1):
    wid = lax.axis_index("s") * NC + lax.axis_index("c")
    base = wid * EPW
    pltpu.sync_copy(eidx_hbm.at[pl.ds(base, EPW)], eidx_v)
    pltpu.async_copy(adje_hbm.at[eidx_v], ae0_v.at[pl.ds(0, EPW)], sem0)
    pltpu.async_copy(adjr_hbm.at[eidx_v], ar0_v.at[pl.ds(0, EPW)], sem0)
    pltpu.async_copy(traw_hbm.at[eidx_v], h_v, sem0)
    pltpu.make_async_copy(adje_hbm.at[eidx_v], ae0_v.at[pl.ds(0, EPW)],
                          sem0).wait()
    pltpu.make_async_copy(adjr_hbm.at[eidx_v], ar0_v.at[pl.ds(0, EPW)],
                          sem0).wait()

    @pl.loop(0, EPW)
    def flatten1(p):
        ne1_v[pl.ds(K * p, K)] = ae0_v[p, :]
        nr1_v[pl.ds(K * p, K)] = ar0_v[p, :]
    pltpu.sync_copy(ne1_v, ne1_out.at[pl.ds(base * K, EPW * K)])
    pltpu.sync_copy(nr1_v, nr1_out.at[pl.ds(base * K, EPW * K)])
    pltpu.make_async_copy(traw_hbm.at[eidx_v], h_v, sem0).wait()
    pltpu.sync_copy(h_v, h_out.at[pl.ds(base, EPW)])

    bufs = ((ae0_v, ar0_v, t10_v, sem0), (ae1_v, ar1_v, t11_v, sem1))

    def fire(c, bf):
        ae, ar, t1, sem = bf
        idx = ne1_v.at[pl.ds(128 * c, 128)]
        pltpu.async_copy(adje_hbm.at[idx], ae, sem)
        pltpu.async_copy(adjr_hbm.at[idx], ar, sem)
        pltpu.async_copy(traw_hbm.at[idx], t1, sem)

    def wait(c, bf):
        ae, ar, t1, sem = bf
        idx = ne1_v.at[pl.ds(128 * c, 128)]
        pltpu.make_async_copy(adje_hbm.at[idx], ae, sem).wait()
        pltpu.make_async_copy(adjr_hbm.at[idx], ar, sem).wait()
        pltpu.make_async_copy(traw_hbm.at[idx], t1, sem).wait()

    fire(0, bufs[0])
    for c in range(4):
        bf = bufs[c % 2]
        wait(c, bf)
        if c + 1 < 4:
            fire(c + 1, bufs[(c + 1) % 2])
        ae, ar, t1, _ = bf

        @pl.loop(0, 128)
        def flatten2(p, _ae=ae, _ar=ar):
            ne2_v[pl.ds(K * p, K)] = _ae[p, :]
            nr2_v[pl.ds(K * p, K)] = _ar[p, :]
        row0 = (base * K + 128 * c)
        pltpu.sync_copy(ne2_v, ne2_out.at[pl.ds(row0 * K, 128 * K)])
        pltpu.sync_copy(nr2_v, nr2_out.at[pl.ds(row0 * K, 128 * K)])
        pltpu.sync_copy(t1, t1_out.at[pl.ds(row0, 128)])


def _phase_a(entity_idx, adje, adjr, traw):
    mesh = plsc.VectorSubcoreMesh(core_axis_name="c", subcore_axis_name="s",
                                  num_cores=NC, num_subcores=NS)
    fn = pl.kernel(
        _phase_a_body,
        out_type=[
            jax.ShapeDtypeStruct((B * K,), jnp.int32),
            jax.ShapeDtypeStruct((B * K,), jnp.int32),
            jax.ShapeDtypeStruct((B * KK,), jnp.int32),
            jax.ShapeDtypeStruct((B * KK,), jnp.int32),
            jax.ShapeDtypeStruct((B, D), jnp.float32),
            jax.ShapeDtypeStruct((B * K, D), jnp.float32),
        ],
        mesh=mesh,
        scratch_types=[
            pltpu.VMEM((EPW,), jnp.int32),
            pltpu.VMEM((128, K), jnp.int32),
            pltpu.VMEM((128, K), jnp.int32),
            pltpu.VMEM((128, K), jnp.int32),
            pltpu.VMEM((128, K), jnp.int32),
            pltpu.VMEM((EPW * K,), jnp.int32),
            pltpu.VMEM((EPW * K,), jnp.int32),
            pltpu.VMEM((128 * K,), jnp.int32),
            pltpu.VMEM((128 * K,), jnp.int32),
            pltpu.VMEM((EPW, D), jnp.float32),
            pltpu.VMEM((128, D), jnp.float32),
            pltpu.VMEM((128, D), jnp.float32),
            pltpu.SemaphoreType.DMA,
            pltpu.SemaphoreType.DMA,
        ],
        compiler_params=pltpu.CompilerParams(
            needs_layout_passes=False, use_tc_tiling_on_sc=False),
    )
    return fn(entity_idx, adje, adjr, traw)


def _softmax_weights(rel_ref, at_ref, w_ref):
    m = jnp.full((L,), -3.0e38, jnp.float32)
    for g in range(KK // L):
        rv = rel_ref[pl.ds(L * g, L)]
        av = plsc.load_gather(at_ref, [rv])
        w_ref[pl.ds(L * g, L)] = av
        m = jnp.maximum(m, av)
    mm = jnp.full((L,), lax.reduce_max(m, axes=(0,)), jnp.float32)
    s = jnp.zeros((L,), jnp.float32)
    for g in range(KK // L):
        ev = jnp.exp(w_ref[pl.ds(L * g, L)] - mm)
        w_ref[pl.ds(L * g, L)] = ev
        s = s + ev
    sb = jnp.full((L,), lax.reduce_sum(s, axes=(0,)), jnp.float32)
    for g in range(KK // L):
        w_ref[pl.ds(L * g, L)] = w_ref[pl.ds(L * g, L)] / sb


def _accumulate(rows_ref, w_ref, half, accs):
    @pl.loop(0, 128, init_carry=accs, unroll=4)
    def row_body(r, accs):
        wbr = plsc.load_gather(
            w_ref, [jnp.full((L,), 128 * half + r, jnp.int32)])
        return tuple(a + wbr * rows_ref[r, pl.ds(L * j, L)]
                     for j, a in enumerate(accs))
    return row_body


def _phase_c_body(boff, nb, ne2_hbm, nr2_hbm, at1f_hbm, tn_hbm, s2_hbm,
                  idx0_v, idx1_v, rel0_v, rel1_v, at0_v, at1_v,
                  w0_v, w1_v, r00_v, r01_v, r10_v, r11_v, acc_v,
                  semi0, semi1, semr00, semr01, semr10, semr11):
    wid = lax.axis_index("s") * NC + lax.axis_index("c")
    eph = nb // NW
    base = boff + wid * eph
    NP = eph // 2

    def stage(b, idx_v, rel_v, at_v, sem):
        c0 = pltpu.async_copy(ne2_hbm.at[pl.ds(b * KK, KK)], idx_v, sem)
        c1 = pltpu.async_copy(nr2_hbm.at[pl.ds(b * KK, KK)], rel_v, sem)
        c2 = pltpu.async_copy(
            at1f_hbm.at[pl.ds((b - boff) * N_REL, N_REL)], at_v, sem)
        return c0, c1, c2

    def wait_stage(b, idx_v, rel_v, at_v, sem):
        pltpu.make_async_copy(ne2_hbm.at[pl.ds(b * KK, KK)], idx_v, sem).wait()
        pltpu.make_async_copy(nr2_hbm.at[pl.ds(b * KK, KK)], rel_v, sem).wait()
        pltpu.make_async_copy(
            at1f_hbm.at[pl.ds((b - boff) * N_REL, N_REL)], at_v, sem).wait()

    def fire_rows(idx_v, ra_v, rb_v, sa, sb):
        pltpu.async_copy(tn_hbm.at[idx_v.at[pl.ds(0, 128)]], ra_v, sa)
        pltpu.async_copy(tn_hbm.at[idx_v.at[pl.ds(128, 128)]], rb_v, sb)

    def wait_rows(idx_v, ra_v, rb_v, sa, sb):
        pltpu.make_async_copy(tn_hbm.at[idx_v.at[pl.ds(0, 128)]], ra_v,
                              sa).wait()
        pltpu.make_async_copy(tn_hbm.at[idx_v.at[pl.ds(128, 128)]], rb_v,
                              sb).wait()

    def reduce_store(b, ra_v, rb_v, w_v):
        accs = tuple(jnp.zeros((L,), jnp.float32) for _ in range(D // L))
        accs = _accumulate(ra_v, w_v, 0, accs)
        accs = _accumulate(rb_v, w_v, 1, accs)
        for j in range(D // L):
            acc_v[pl.ds(L * j, L)] = accs[j]
        pltpu.sync_copy(acc_v, s2_hbm.at[pl.ds((b - boff) * D, D)])

    stage(base, idx0_v, rel0_v, at0_v, semi0)
    wait_stage(base, idx0_v, rel0_v, at0_v, semi0)
    fire_rows(idx0_v, r00_v, r01_v, semr00, semr01)
    stage(base + 1, idx1_v, rel1_v, at1_v, semi1)

    @pl.loop(0, NP)
    def pair(t):
        b0 = base + 2 * t
        wait_stage(b0 + 1, idx1_v, rel1_v, at1_v, semi1)
        fire_rows(idx1_v, r10_v, r11_v, semr10, semr11)
        _softmax_weights(rel0_v, at0_v, w0_v)
        wait_rows(idx0_v, r00_v, r01_v, semr00, semr01)

        @pl.when(t + 1 < NP)
        def _():
            stage(b0 + 2, idx0_v, rel0_v, at0_v, semi0)
        reduce_store(b0, r00_v, r01_v, w0_v)

        _softmax_weights(rel1_v, at1_v, w1_v)

        @pl.when(t + 1 < NP)
        def _():
            wait_stage(b0 + 2, idx0_v, rel0_v, at0_v, semi0)
            fire_rows(idx0_v, r00_v, r01_v, semr00, semr01)
        wait_rows(idx1_v, r10_v, r11_v, semr10, semr11)

        @pl.when(t + 1 < NP)
        def _():
            stage(b0 + 3, idx1_v, rel1_v, at1_v, semi1)
        reduce_store(b0 + 1, r10_v, r11_v, w1_v)


def _phase_c(ne2f, nr2f, at1f, tn, boff=0, nb=B):
    mesh = plsc.VectorSubcoreMesh(core_axis_name="c", subcore_axis_name="s",
                                  num_cores=NC, num_subcores=NS)
    fn = pl.kernel(
        functools.partial(_phase_c_body, boff, nb),
        out_type=jax.ShapeDtypeStruct((nb * D,), jnp.float32),
        mesh=mesh,
        scratch_types=[
            pltpu.VMEM((KK,), jnp.int32),
            pltpu.VMEM((KK,), jnp.int32),
            pltpu.VMEM((KK,), jnp.int32),
            pltpu.VMEM((KK,), jnp.int32),
            pltpu.VMEM((N_REL,), jnp.float32),
            pltpu.VMEM((N_REL,), jnp.float32),
            pltpu.VMEM((KK,), jnp.float32),
            pltpu.VMEM((KK,), jnp.float32),
            pltpu.VMEM((128, D), jnp.float32),
            pltpu.VMEM((128, D), jnp.float32),
            pltpu.VMEM((128, D), jnp.float32),
            pltpu.VMEM((128, D), jnp.float32),
            pltpu.VMEM((D,), jnp.float32),
            pltpu.SemaphoreType.DMA,
            pltpu.SemaphoreType.DMA,
            pltpu.SemaphoreType.DMA,
            pltpu.SemaphoreType.DMA,
            pltpu.SemaphoreType.DMA,
            pltpu.SemaphoreType.DMA,
        ],
        compiler_params=pltpu.CompilerParams(needs_layout_passes=False),
    )
    return fn(ne2f, nr2f, at1f, tn)


_BS_B = 128


def _leaky(x):
    return jnp.where(x >= 0.0, x, 0.2 * x)


def _sigmoid(x):
    return 1.0 / (1.0 + jnp.exp(-x))


def _att_table(base, a1t, ra1r, a2, a3t, bs):
    pre = jnp.dot(base, a1t, preferred_element_type=jnp.float32)
    x = jnp.maximum(pre[:, None, :] + ra1r[None, :, :], 0.0)
    x = x.reshape(bs * N_REL, D)
    x = jnp.maximum(jnp.dot(x, a2, preferred_element_type=jnp.float32), 0.0)
    att = jnp.sum(x * a3t, axis=1)
    return _sigmoid(att).reshape(bs, N_REL)


def _onehot_pick(tbl, idx):
    out = jnp.zeros(idx.shape, jnp.float32)
    for r in range(N_REL):
        out = out + jnp.where(idx == r, tbl[:, r:r + 1], 0.0)
    return out


def _softmax_last(x):
    m = jnp.max(x, axis=-1, keepdims=True)
    e = jnp.exp(x - m)
    return e / jnp.sum(e, axis=-1, keepdims=True)


def _phase_b_body(h_ref, t1_ref, nr1_ref, rel_ref,
                  a1_ref, a2_ref, a3t_ref, wx00_ref, wx01_ref,
                  wxb00_ref, wxb01_ref, w1w_ref, w1b_ref, w2w_ref, w2b_ref,
                  at1_ref, base1_ref, ee0_ref):
    bs = _BS_B
    h = _rownorm(h_ref[...])
    t1 = _rownorm(t1_ref[...]).reshape(bs, K, D)
    reln = _rownorm(rel_ref[...])
    a1t = a1_ref[:D, :]
    ra1r = jnp.dot(reln, a1_ref[D:, :], preferred_element_type=jnp.float32)
    a2 = a2_ref[...]
    a3t = a3t_ref[...]

    base1 = jnp.sum(t1, axis=1)
    base1_ref[...] = base1

    at0 = _att_table(h, a1t, ra1r, a2, a3t, bs)
    at1 = _att_table(base1, a1t, ra1r, a2, a3t, bs)

    att1 = _onehot_pick(at0, nr1_ref[...])
    attn1 = _softmax_last(att1)
    vec1 = jnp.zeros((bs, D), jnp.float32)
    for k in range(K):
        vec1 = vec1 + attn1[:, k:k + 1] * t1[:, k, :]
    e0 = jnp.dot(vec1, wx00_ref[...], preferred_element_type=jnp.float32) + wxb00_ref[...]
    e1 = jnp.dot(vec1, wx01_ref[...], preferred_element_type=jnp.float32) + wxb01_ref[...]
    vec = _leaky(jnp.concatenate([e0, e1], axis=-1))
    hexp = jnp.concatenate([h, h], axis=-1)
    ee0 = (_leaky(jnp.dot(hexp + vec, w1w_ref[...], preferred_element_type=jnp.float32) + w1b_ref[...])
           + _leaky(jnp.dot(hexp * vec, w2w_ref[...], preferred_element_type=jnp.float32) + w2b_ref[...]))
    ee0_ref[...] = ee0
    at1_ref[...] = at1


def _phase_b(h_norm, t1n, nr1, relation_table,
             A1, A2, A3, Wx, Wx_b, W1_w, W1_b, W2_w, W2_b):
    bs = _BS_B
    nb = h_norm.shape[0]
    grid = nb // bs
    full = lambda shape: pl.BlockSpec(shape, lambda i: tuple(0 for _ in shape))
    out = pl.pallas_call(
        _phase_b_body,
        grid=(grid,),
        in_specs=[
            pl.BlockSpec((bs, D), lambda i: (i, 0)),
            pl.BlockSpec((bs * K, D), lambda i: (i, 0)),
            pl.BlockSpec((bs, K), lambda i: (i, 0)),
            full((N_REL, D)),
            full((2 * D, D)),
            full((D, D)),
            full((1, D)),
            full((D, D)),
            full((D, D)),
            full((1, D)),
            full((1, D)),
            full((2 * D, 2 * D)),
            full((1, 2 * D)),
            full((2 * D, 2 * D)),
            full((1, 2 * D)),
        ],
        out_specs=[
            pl.BlockSpec((bs, N_REL), lambda i: (i, 0)),
            pl.BlockSpec((bs, D), lambda i: (i, 0)),
            pl.BlockSpec((bs, 2 * D), lambda i: (i, 0)),
        ],
        out_shape=[
            jax.ShapeDtypeStruct((nb, N_REL), jnp.float32),
            jax.ShapeDtypeStruct((nb, D), jnp.float32),
            jax.ShapeDtypeStruct((nb, 2 * D), jnp.float32),
        ],
    )(h_norm, t1n, nr1, relation_table,
      A1, A2, A3.reshape(1, D), Wx[0, 0], Wx[0, 1],
      Wx_b[0, 0].reshape(1, D), Wx_b[0, 1].reshape(1, D),
      W1_w, W1_b.reshape(1, 2 * D), W2_w, W2_b.reshape(1, 2 * D))
    return out


_BS_D = 128


def _phase_d_body(s2_ref, base1_ref, ee0_ref, h_ref,
                  wx10_ref, wx11_ref, wxb10_ref, wxb11_ref,
                  w1w_ref, w1b_ref, w2w_ref, w2b_ref, out_ref):
    s2 = s2_ref[...]
    base1 = base1_ref[...]
    e0 = jnp.dot(s2, wx10_ref[...], preferred_element_type=jnp.float32) + wxb10_ref[...]
    e1 = jnp.dot(s2, wx11_ref[...], preferred_element_type=jnp.float32) + wxb11_ref[...]
    vec = _leaky(jnp.concatenate([e0, e1], axis=-1))
    hexp = jnp.concatenate([base1, base1], axis=-1)
    ee1 = (_leaky(jnp.dot(hexp + vec, w1w_ref[...], preferred_element_type=jnp.float32) + w1b_ref[...])
           + _leaky(jnp.dot(hexp * vec, w2w_ref[...], preferred_element_type=jnp.float32) + w2b_ref[...]))
    out_ref[...] = jnp.concatenate([ee1, ee0_ref[...], _rownorm(h_ref[...])],
                                   axis=-1)


def _phase_d(s2, base1, ee0, h_norm, Wx, Wx_b, W1_w, W1_b, W2_w, W2_b):
    bs = _BS_D
    nb = s2.shape[0]
    grid = nb // bs
    full = lambda shape: pl.BlockSpec(shape, lambda i: tuple(0 for _ in shape))
    return pl.pallas_call(
        _phase_d_body,
        grid=(grid,),
        in_specs=[
            pl.BlockSpec((bs, D), lambda i: (i, 0)),
            pl.BlockSpec((bs, D), lambda i: (i, 0)),
            pl.BlockSpec((bs, 2 * D), lambda i: (i, 0)),
            pl.BlockSpec((bs, D), lambda i: (i, 0)),
            full((D, D)),
            full((D, D)),
            full((1, D)),
            full((1, D)),
            full((2 * D, 2 * D)),
            full((1, 2 * D)),
            full((2 * D, 2 * D)),
            full((1, 2 * D)),
        ],
        out_specs=pl.BlockSpec((bs, 5 * D), lambda i: (i, 0)),
        out_shape=jax.ShapeDtypeStruct((nb, 5 * D), jnp.float32),
    )(s2, base1, ee0, h_norm, Wx[1, 0], Wx[1, 1],
      Wx_b[1, 0].reshape(1, D), Wx_b[1, 1].reshape(1, D),
      W1_w, W1_b.reshape(1, 2 * D), W2_w, W2_b.reshape(1, 2 * D))


def kernel(entity_idx, adj_entity, adj_relation, entity_table, relation_table,
           A1, A2, A3, Wx, Wx_b, W1_w, W1_b, W2_w, W2_b):
    tn = _normalize_table(entity_table, blk=4000)
    ne1f, nr1f, ne2f, nr2f, h_raw, t1_raw = _phase_a(
        entity_idx, adj_entity, adj_relation, entity_table)
    H = B // 2
    nr1r = nr1f.reshape(B, K)
    outs = []
    parts = []
    for half in range(2):
        sl = slice(half * H, (half + 1) * H)
        at1, base1, ee0 = _phase_b(
            h_raw[sl], t1_raw[half * H * K:(half + 1) * H * K], nr1r[sl],
            relation_table, A1, A2, A3, Wx, Wx_b, W1_w, W1_b, W2_w, W2_b)
        parts.append((sl, at1, base1, ee0))
    for half, (sl, at1, base1, ee0) in enumerate(parts):
        s2 = _phase_c(ne2f, nr2f, at1.reshape(-1), tn,
                      boff=half * H, nb=H).reshape(H, D)
        outs.append(_phase_d(s2, base1, ee0, h_raw[sl],
                             Wx, Wx_b, W1_w, W1_b, W2_w, W2_b))
    return jnp.concatenate(outs, axis=0)
